# baseline (speedup 1.0000x reference)
import jax
import jax.numpy as jnp
from jax.experimental import pallas as pl

_K = 8192
_TT = 1024


def _lfq_tile(x_ref, cb_ref, out_ref):
    xt = x_ref[0]
    cb = cb_ref[...]
    logits = jax.lax.dot_general(
        xt, cb, (((1,), (1,)), ((), ())), preferred_element_type=jnp.float32
    )
    nc = _K // 128
    level = [
        (logits[:, c * 128 : (c + 1) * 128], jnp.full((_TT, 128), c, jnp.int32))
        for c in range(nc)
    ]
    while len(level) > 1:
        nxt = []
        for a, b in zip(level[0::2], level[1::2]):
            pred = b[0] > a[0]
            nxt.append((jnp.where(pred, b[0], a[0]), jnp.where(pred, b[1], a[1])))
        level = nxt
    best_val, best_c = level[0]
    m = jnp.max(best_val, axis=1, keepdims=True)
    lane = jax.lax.broadcasted_iota(jnp.int32, (_TT, 128), 1)
    k_full = best_c * 128 + lane
    idx = jnp.min(jnp.where(best_val == m, k_full, _K), axis=1)
    out_ref[0, 0, 0, :] = idx


def kernel(x, codebook):
    B, T, D = x.shape
    nt = T // _TT
    out = pl.pallas_call(
        _lfq_tile,
        grid=(B, nt),
        in_specs=[
            pl.BlockSpec((1, _TT, D), lambda b, t: (b, t, 0)),
            pl.BlockSpec((_K, D), lambda b, t: (0, 0)),
        ],
        out_specs=pl.BlockSpec((1, 1, 1, _TT), lambda b, t: (b, t, 0, 0)),
        out_shape=jax.ShapeDtypeStruct((B, nt, 1, _TT), jnp.int32),
    )(x, codebook)
    return out.reshape(B, T), jnp.asarray(0.0, dtype=jnp.float32)


# speedup vs baseline: 1.0625x; 1.0625x over previous
import jax
import jax.numpy as jnp
from jax.experimental import pallas as pl

_K = 8192
_TT = 1024


def _lfq_tile(x_ref, cb_ref, out_ref):
    xt = x_ref[0]
    cb = cb_ref[...]
    logits = jax.lax.dot_general(
        xt, cb, (((1,), (1,)), ((), ())), preferred_element_type=jnp.float32
    )
    nc = _K // 128
    best_val = logits[:, 0:128]
    best_c = jnp.zeros((_TT, 128), jnp.float32)
    for c in range(1, nc):
        chunk = logits[:, c * 128 : (c + 1) * 128]
        pred = chunk > best_val
        best_val = jnp.where(pred, chunk, best_val)
        best_c = jnp.where(pred, jnp.float32(c), best_c)
    m = jnp.max(best_val, axis=1, keepdims=True)
    lane = jax.lax.broadcasted_iota(jnp.int32, (_TT, 128), 1).astype(jnp.float32)
    k_full = best_c * 128.0 + lane
    cand = jnp.where(best_val == m, k_full, jnp.float32(_K))
    idx = jnp.min(cand, axis=1).astype(jnp.int32)
    out_ref[0, 0, 0, :] = idx


def kernel(x, codebook):
    B, T, D = x.shape
    nt = T // _TT
    out = pl.pallas_call(
        _lfq_tile,
        grid=(B, nt),
        in_specs=[
            pl.BlockSpec((1, _TT, D), lambda b, t: (b, t, 0)),
            pl.BlockSpec((_K, D), lambda b, t: (0, 0)),
        ],
        out_specs=pl.BlockSpec((1, 1, 1, _TT), lambda b, t: (b, t, 0, 0)),
        out_shape=jax.ShapeDtypeStruct((B, nt, 1, _TT), jnp.int32),
    )(x, codebook)
    return out.reshape(B, T), jnp.asarray(0.0, dtype=jnp.float32)


# speedup vs baseline: 1.0706x; 1.0077x over previous
import jax
import jax.numpy as jnp
from jax.experimental import pallas as pl

_K = 8192
_TT = 512


def _lfq_tile(x_ref, cb_ref, out_ref):
    xt = x_ref[0]
    cb = cb_ref[...]
    logits = jax.lax.dot_general(
        xt, cb, (((1,), (1,)), ((), ())), preferred_element_type=jnp.float32
    )
    nc = _K // 128
    best_val = logits[:, 0:128]
    best_c = jnp.zeros((_TT, 128), jnp.float32)
    for c in range(1, nc):
        chunk = logits[:, c * 128 : (c + 1) * 128]
        pred = chunk > best_val
        best_val = jnp.where(pred, chunk, best_val)
        best_c = jnp.where(pred, jnp.float32(c), best_c)
    m = jnp.max(best_val, axis=1, keepdims=True)
    lane = jax.lax.broadcasted_iota(jnp.int32, (_TT, 128), 1).astype(jnp.float32)
    k_full = best_c * 128.0 + lane
    cand = jnp.where(best_val == m, k_full, jnp.float32(_K))
    idx = jnp.min(cand, axis=1).astype(jnp.int32)
    out_ref[0, 0, 0, :] = idx


def kernel(x, codebook):
    B, T, D = x.shape
    nt = T // _TT
    out = pl.pallas_call(
        _lfq_tile,
        grid=(B, nt),
        in_specs=[
            pl.BlockSpec((1, _TT, D), lambda b, t: (b, t, 0)),
            pl.BlockSpec((_K, D), lambda b, t: (0, 0)),
        ],
        out_specs=pl.BlockSpec((1, 1, 1, _TT), lambda b, t: (b, t, 0, 0)),
        out_shape=jax.ShapeDtypeStruct((B, nt, 1, _TT), jnp.int32),
    )(x, codebook)
    return out.reshape(B, T), jnp.asarray(0.0, dtype=jnp.float32)
